# baseline (speedup 1.0000x reference)
import functools

import jax
import jax.numpy as jnp
from jax import lax
from jax.experimental import pallas as pl
from jax.experimental.pallas import tpu as pltpu
from jax.experimental.pallas import tpu_sc as plsc

N = 10000
E = 160000
HEADS = 8
HID = 32
NCLS = 40

NC = 2
NS = 16
LANES = 16
NW = NC * NS
NRANGES = 64
NPR = 157
NPAD = NRANGES * NPR
CAP = 4096
SCN = 4096
CHUNK = 64

_f32 = jnp.float32
_i32 = jnp.int32


def _mesh():
    return plsc.VectorSubcoreMesh(
        core_axis_name="c", subcore_axis_name="s",
        num_cores=NC, num_subcores=NS)


_SC_PARAMS = pltpu.CompilerParams(needs_layout_passes=False)


def _compact_body(src_hbm, dst_hbm, srcl_hbm, dstl_hbm, cnt_hbm,
                  schunk, dchunk, srcl0, dstl0, srcl1, dstl1, cnt_v):
    w = lax.axis_index("s") * NC + lax.axis_index("c")
    lo0 = w * 2 * NPR
    lo1 = lo0 + NPR
    hi1 = lo1 + NPR

    def zfill(i, _):
        z = jnp.zeros((LANES,), _i32)
        f = jnp.full((LANES,), NPR, _i32)
        srcl0[pl.ds(i * LANES, LANES)] = z
        dstl0[pl.ds(i * LANES, LANES)] = f
        srcl1[pl.ds(i * LANES, LANES)] = z
        dstl1[pl.ds(i * LANES, LANES)] = f
        return 0
    lax.fori_loop(0, CAP // LANES, zfill, 0)

    def chunk_body(ch, offs):
        pltpu.sync_copy(src_hbm.at[pl.ds(ch * SCN, SCN)], schunk)
        pltpu.sync_copy(dst_hbm.at[pl.ds(ch * SCN, SCN)], dchunk)

        def grp(g, offs):
            off0, off1 = offs
            d = dchunk[pl.ds(g * LANES, LANES)]
            s = schunk[pl.ds(g * LANES, LANES)]
            in0 = (d >= lo0) & (d < lo1)
            in1 = (d >= lo1) & (d < hi1)
            c0 = plsc.cumsum(in0.astype(_i32))
            c1 = plsc.cumsum(in1.astype(_i32))
            p0 = jnp.minimum(off0 + c0 - 1, CAP - 1)
            p1 = jnp.minimum(off1 + c1 - 1, CAP - 1)
            plsc.store_scatter(srcl0, [p0], s, mask=in0)
            plsc.store_scatter(dstl0, [p0], d - lo0, mask=in0)
            plsc.store_scatter(srcl1, [p1], s, mask=in1)
            plsc.store_scatter(dstl1, [p1], d - lo1, mask=in1)
            return (off0 + jnp.max(c0), off1 + jnp.max(c1))
        return lax.fori_loop(0, SCN // LANES, grp, offs)

    z = jnp.array(0, _i32)
    off0, off1 = lax.fori_loop(0, E // SCN, chunk_body, (z, z))

    for r, (off, sv, dv) in enumerate(((off0, srcl0, dstl0),
                                       (off1, srcl1, dstl1))):
        rid = w * 2 + r
        off16 = jnp.minimum((off + LANES - 1) // LANES * LANES, CAP)
        for q in range(128 // LANES):
            cnt_v[pl.ds(q * LANES, LANES)] = jnp.full((LANES,), off16, _i32)
        pltpu.sync_copy(cnt_v, cnt_hbm.at[rid])
        pltpu.sync_copy(sv, srcl_hbm.at[rid])
        pltpu.sync_copy(dv, dstl_hbm.at[rid])


_compact = pl.kernel(
    _compact_body,
    out_type=(
        jax.ShapeDtypeStruct((NRANGES, CAP), _i32),
        jax.ShapeDtypeStruct((NRANGES, CAP), _i32),
        jax.ShapeDtypeStruct((NRANGES, 128), _i32),
    ),
    mesh=_mesh(),
    compiler_params=_SC_PARAMS,
    scratch_types=[
        pltpu.VMEM((SCN,), _i32),
        pltpu.VMEM((SCN,), _i32),
        pltpu.VMEM((CAP,), _i32),
        pltpu.VMEM((CAP,), _i32),
        pltpu.VMEM((CAP,), _i32),
        pltpu.VMEM((CAP,), _i32),
        pltpu.VMEM((128,), _i32),
    ],
)


def _gat_body(heads, dh, d_store,
              xlp_hbm, xr_hbm, srcl_hbm, dstl_hbm, cnt_hbm, att_hbm, b_hbm,
              out_hbm,
              srcl_v, dstl_v, xl_tile, out_un, bufA, a_buf, denom_v,
              att_v, b_v, cnt_v, sem0):
    d_feat = heads * dh
    nvr = d_feat // LANES
    w = lax.axis_index("s") * NC + lax.axis_index("c")
    iota = lax.iota(_i32, LANES)

    pltpu.sync_copy(att_hbm, att_v)
    pltpu.sync_copy(b_hbm, b_v)

    def do_range(r, _):
        rid = w * 2 + r
        lo = rid * NPR
        pltpu.sync_copy(cnt_hbm.at[rid], cnt_v)
        cnt16 = jnp.max(cnt_v[pl.ds(0, LANES)])
        nch = (cnt16 + CHUNK - 1) // CHUNK
        pltpu.sync_copy(srcl_hbm.at[rid], srcl_v)
        pltpu.sync_copy(dstl_hbm.at[rid], dstl_v)
        pltpu.sync_copy(xlp_hbm.at[pl.ds(lo * d_feat, (NPR + 1) * d_feat)],
                        xl_tile)

        def zinit(i, _):
            out_un[pl.ds(i * LANES, LANES)] = jnp.zeros((LANES,), _f32)
            return 0
        lax.fori_loop(0, (NPR + 1) * d_feat // LANES, zinit, 0)

        def zden(i, _):
            denom_v[pl.ds(i * LANES, LANES)] = jnp.zeros((LANES,), _f32)
            return 0
        lax.fori_loop(0, (160 * heads) // LANES, zden, 0)

        def chunk_body(ci, _):
            coff = ci * CHUNK
            cp = pltpu.async_copy(
                xr_hbm.at[srcl_v.at[pl.ds(coff, CHUNK)]], bufA, sem0)
            cp.wait()
            ngc = jnp.minimum(CHUNK // LANES,
                              (cnt16 - coff + LANES - 1) // LANES)

            def grp(g, _):
                rows = g * LANES + iota
                dv = dstl_v[pl.ds(coff + g * LANES, LANES)]
                dvb = dv * heads
                xl_base = dv * d_feat
                dvf = dv * d_feat

                def hloop(h, _):
                    acc = jnp.zeros((LANES,), _f32)
                    for dd in range(dh):
                        c = h * dh + dd
                        xlv = plsc.load_gather(xl_tile, [xl_base + c])
                        xrv = plsc.load_gather(bufA, [rows,
                                                      jnp.full((LANES,), c,
                                                               _i32)])
                        sv = xlv + xrv
                        sv = jnp.maximum(sv, sv * 0.2)
                        av = plsc.load_gather(att_v,
                                              [jnp.full((LANES,), c, _i32)])
                        acc = acc + sv * av
                    a = jnp.exp(jnp.minimum(acc, 30.0))
                    a_buf[pl.ds(h * LANES, LANES)] = a
                    plsc.addupdate_scatter(denom_v, [dvb + h], a)
                    return 0
                lax.fori_loop(0, heads, hloop, 0)

                avs = [a_buf[pl.ds(h * LANES, LANES)] for h in range(heads)]
                for cb in range(0, d_feat, 8):
                    xs = [plsc.load_gather(
                        bufA, [rows, jnp.full((LANES,), c, _i32)])
                        for c in range(cb, cb + 8)]
                    for q, c in enumerate(range(cb, cb + 8)):
                        plsc.addupdate_scatter(
                            out_un, [dvf + c], xs[q] * avs[c // dh])
                return 0
            lax.fori_loop(0, ngc, grp, 0)
            return 0
        lax.fori_loop(0, nch, chunk_body, 0)

        def norm(i, _):
            rb = i * d_feat
            for h in range(heads):
                dn = plsc.load_gather(
                    denom_v, [jnp.full((LANES,), i * heads + h, _i32)])
                rcp = 1.0 / (dn + 1e-16)
                for j in range(dh // LANES):
                    sl = pl.ds(rb + h * dh + j * LANES, LANES)
                    out_un[sl] = (out_un[sl] * rcp
                                  + b_v[pl.ds(h * dh + j * LANES, LANES)])
            return 0
        lax.fori_loop(0, NPR, norm, 0)

        pltpu.sync_copy(out_un.at[pl.ds(0, NPR * d_feat)],
                        out_hbm.at[pl.ds(lo * d_feat, NPR * d_feat)])
        return 0

    lax.fori_loop(0, 2, do_range, 0)


@functools.cache
def _gat_sc(heads, dh, d_store):
    d_feat = heads * dh
    return pl.kernel(
        functools.partial(_gat_body, heads, dh, d_store),
        out_type=jax.ShapeDtypeStruct((NPAD * d_feat,), _f32),
        mesh=_mesh(),
        compiler_params=_SC_PARAMS,
        scratch_types=[
            pltpu.VMEM((CAP,), _i32),
            pltpu.VMEM((CAP,), _i32),
            pltpu.VMEM(((NPR + 1) * d_feat,), _f32),
            pltpu.VMEM(((NPR + 1) * d_feat,), _f32),
            pltpu.VMEM((CHUNK, d_store), _f32),
            pltpu.VMEM((heads * LANES,), _f32),
            pltpu.VMEM((160 * heads,), _f32),
            pltpu.VMEM((d_feat,), _f32),
            pltpu.VMEM((d_feat,), _f32),
            pltpu.VMEM((128,), _i32),
            pltpu.SemaphoreType.DMA,
        ],
    )


def _mm2_body(elu, x_ref, wl_ref, wr_ref, ol_ref, or_ref):
    xv = x_ref[...]
    if elu:
        xv = jnp.where(xv > 0, xv, jnp.exp(jnp.minimum(xv, 0.0)) - 1.0)
    ol_ref[...] = jnp.dot(xv, wl_ref[...], preferred_element_type=_f32)
    or_ref[...] = jnp.dot(xv, wr_ref[...], preferred_element_type=_f32)


def _matmul2(x, wl, wr, elu, block_rows=1000):
    n, k = x.shape
    m = wl.shape[1]
    grid = (n // block_rows,)
    return pl.pallas_call(
        functools.partial(_mm2_body, elu),
        grid=grid,
        in_specs=[
            pl.BlockSpec((block_rows, k), lambda i: (i, 0)),
            pl.BlockSpec((k, m), lambda i: (0, 0)),
            pl.BlockSpec((k, m), lambda i: (0, 0)),
        ],
        out_specs=[
            pl.BlockSpec((block_rows, m), lambda i: (i, 0)),
            pl.BlockSpec((block_rows, m), lambda i: (i, 0)),
        ],
        out_shape=[
            jax.ShapeDtypeStruct((n, m), _f32),
            jax.ShapeDtypeStruct((n, m), _f32),
        ],
    )(x, wl, wr)


def _pool_body(h_ref, o_ref):
    pooled = jnp.sum(h_ref[...], axis=0, keepdims=True) / N
    colid = lax.broadcasted_iota(_i32, (1, 48), 1)
    mask = colid < NCLS
    pm = jnp.where(mask, pooled, -1e30)
    mx = jnp.max(pm, axis=1, keepdims=True)
    ex = jnp.where(mask, jnp.exp(pm - mx), 0.0)
    lse = jnp.log(jnp.sum(ex, axis=1, keepdims=True)) + mx
    o_ref[...] = jnp.broadcast_to(pooled - lse, (8, 48))


def _pool(h3):
    return pl.pallas_call(
        _pool_body,
        out_shape=jax.ShapeDtypeStruct((8, 48), _f32),
    )(h3)


def _pad_flat(xl, d_feat):
    return jnp.pad(xl.reshape(-1), (0, (NPAD + 1 - N) * d_feat))


def kernel(x, edge_index, Wl1, Wr1, att1, b1, Wl2, Wr2, att2, b2,
           Wl3, Wr3, att3, b3):
    src = edge_index[0]
    dst = edge_index[1]
    srcl, dstl, cnt = _compact(src, dst)

    xl1, xr1 = _matmul2(x, Wl1, Wr1, elu=False)
    h1 = _gat_sc(HEADS, HID, 256)(_pad_flat(xl1, 256), xr1, srcl, dstl, cnt,
                                  att1.reshape(-1), b1)
    h1 = h1.reshape(NPAD, HEADS * HID)[:N]

    xl2, xr2 = _matmul2(h1, Wl2, Wr2, elu=True)
    h2 = _gat_sc(HEADS, HID, 256)(_pad_flat(xl2, 256), xr2, srcl, dstl, cnt,
                                  att2.reshape(-1), b2)
    h2 = h2.reshape(NPAD, HEADS * HID)[:N]

    wl3 = jnp.pad(Wl3, ((0, 0), (0, 48 - NCLS)))
    wr3 = jnp.pad(Wr3, ((0, 0), (0, 128 - NCLS)))
    att3p = jnp.pad(att3, ((0, 0), (0, 48 - NCLS))).reshape(-1)
    b3p = jnp.pad(b3, (0, 48 - NCLS))
    xl3, xr3 = _matmul2(h2, jnp.pad(wl3, ((0, 0), (0, 80))), wr3, elu=True)
    h3 = _gat_sc(1, 48, 128)(_pad_flat(xl3[:, :48], 48), xr3, srcl, dstl,
                             cnt, att3p, b3p)
    h3 = h3.reshape(NPAD, 48)

    z = _pool(h3)
    return z[0:1, :NCLS]


# speedup vs baseline: 2.7253x; 2.7253x over previous
import functools

import jax
import jax.numpy as jnp
from jax import lax
from jax.experimental import pallas as pl
from jax.experimental.pallas import tpu as pltpu
from jax.experimental.pallas import tpu_sc as plsc

N = 10000
E = 160000
HEADS = 8
HID = 32
NCLS = 40

NC = 2
NS = 16
LANES = 16
NW = NC * NS
NRANGES = 64
NPR = 157
NPAD = NRANGES * NPR
CAP = 4096
SCN = 4096
CHUNK = 64

_f32 = jnp.float32
_i32 = jnp.int32


def _mesh():
    return plsc.VectorSubcoreMesh(
        core_axis_name="c", subcore_axis_name="s",
        num_cores=NC, num_subcores=NS)


_SC_PARAMS = pltpu.CompilerParams(needs_layout_passes=False)


def _compact_body(src_hbm, dst_hbm, srcl_hbm, dstl_hbm, cnt_hbm,
                  schunk, dchunk, srcl0, dstl0, srcl1, dstl1, cnt_v):
    w = lax.axis_index("s") * NC + lax.axis_index("c")
    lo0 = w * 2 * NPR
    lo1 = lo0 + NPR
    hi1 = lo1 + NPR

    def zfill(i, _):
        z = jnp.zeros((LANES,), _i32)
        f = jnp.full((LANES,), NPR, _i32)
        srcl0[pl.ds(i * LANES, LANES)] = z
        dstl0[pl.ds(i * LANES, LANES)] = f
        srcl1[pl.ds(i * LANES, LANES)] = z
        dstl1[pl.ds(i * LANES, LANES)] = f
        return 0
    lax.fori_loop(0, CAP // LANES, zfill, 0)

    def chunk_body(ch, offs):
        pltpu.sync_copy(src_hbm.at[pl.ds(ch * SCN, SCN)], schunk)
        pltpu.sync_copy(dst_hbm.at[pl.ds(ch * SCN, SCN)], dchunk)

        def grp(g, offs):
            off0, off1 = offs
            d = dchunk[pl.ds(g * LANES, LANES)]
            s = schunk[pl.ds(g * LANES, LANES)]
            in0 = (d >= lo0) & (d < lo1)
            in1 = (d >= lo1) & (d < hi1)
            c0 = plsc.cumsum(in0.astype(_i32))
            c1 = plsc.cumsum(in1.astype(_i32))
            p0 = jnp.minimum(off0 + c0 - 1, CAP - 1)
            p1 = jnp.minimum(off1 + c1 - 1, CAP - 1)
            plsc.store_scatter(srcl0, [p0], s, mask=in0)
            plsc.store_scatter(dstl0, [p0], d - lo0, mask=in0)
            plsc.store_scatter(srcl1, [p1], s, mask=in1)
            plsc.store_scatter(dstl1, [p1], d - lo1, mask=in1)
            return (off0 + jnp.max(c0), off1 + jnp.max(c1))
        return lax.fori_loop(0, SCN // LANES, grp, offs)

    z = jnp.array(0, _i32)
    off0, off1 = lax.fori_loop(0, E // SCN, chunk_body, (z, z))

    for r, (off, sv, dv) in enumerate(((off0, srcl0, dstl0),
                                       (off1, srcl1, dstl1))):
        rid = w * 2 + r
        off16 = jnp.minimum((off + LANES - 1) // LANES * LANES, CAP)
        for q in range(128 // LANES):
            cnt_v[pl.ds(q * LANES, LANES)] = jnp.full((LANES,), off16, _i32)
        pltpu.sync_copy(cnt_v, cnt_hbm.at[rid])
        pltpu.sync_copy(sv, srcl_hbm.at[rid])
        pltpu.sync_copy(dv, dstl_hbm.at[rid])


_compact = pl.kernel(
    _compact_body,
    out_type=(
        jax.ShapeDtypeStruct((NRANGES, CAP), _i32),
        jax.ShapeDtypeStruct((NRANGES, CAP), _i32),
        jax.ShapeDtypeStruct((NRANGES, 128), _i32),
    ),
    mesh=_mesh(),
    compiler_params=_SC_PARAMS,
    scratch_types=[
        pltpu.VMEM((SCN,), _i32),
        pltpu.VMEM((SCN,), _i32),
        pltpu.VMEM((CAP,), _i32),
        pltpu.VMEM((CAP,), _i32),
        pltpu.VMEM((CAP,), _i32),
        pltpu.VMEM((CAP,), _i32),
        pltpu.VMEM((128,), _i32),
    ],
)


def _gat_body(heads, dh, d_store,
              xlp_hbm, xr_hbm, srcl_hbm, dstl_hbm, cnt_hbm, att_hbm, b_hbm,
              out_hbm,
              srcl_v, dstl_v, xl_tile, out_un, bufA, a_buf, denom_v,
              att_v, b_v, cnt_v, sem0):
    d_feat = heads * dh
    nvr = d_feat // LANES
    w = lax.axis_index("s") * NC + lax.axis_index("c")
    iota = lax.iota(_i32, LANES)

    pltpu.sync_copy(att_hbm, att_v)
    pltpu.sync_copy(b_hbm, b_v)

    def do_range(r, _):
        rid = w * 2 + r
        lo = rid * NPR
        pltpu.sync_copy(cnt_hbm.at[rid], cnt_v)
        cnt16 = jnp.max(cnt_v[pl.ds(0, LANES)])
        nch = (cnt16 + CHUNK - 1) // CHUNK
        pltpu.sync_copy(srcl_hbm.at[rid], srcl_v)
        pltpu.sync_copy(dstl_hbm.at[rid], dstl_v)
        pltpu.sync_copy(xlp_hbm.at[pl.ds(lo * d_feat, (NPR + 1) * d_feat)],
                        xl_tile)

        def zinit(i, _):
            out_un[pl.ds(i * LANES, LANES)] = jnp.zeros((LANES,), _f32)
            return 0
        lax.fori_loop(0, (NPR + 1) * d_feat // LANES, zinit, 0)

        def zden(i, _):
            denom_v[pl.ds(i * LANES, LANES)] = jnp.zeros((LANES,), _f32)
            return 0
        lax.fori_loop(0, (160 * heads) // LANES, zden, 0)

        def chunk_body(ci, _):
            coff = ci * CHUNK
            cp = pltpu.async_copy(
                xr_hbm.at[srcl_v.at[pl.ds(coff, CHUNK)]], bufA, sem0)
            cp.wait()
            ngc = jnp.minimum(CHUNK // LANES,
                              (cnt16 - coff + LANES - 1) // LANES)

            def grp(g, _):
                rows = g * LANES + iota
                dv = dstl_v[pl.ds(coff + g * LANES, LANES)]
                dvb = dv * heads
                xl_base = dv * d_feat

                def hloop(h, _):
                    acc = jnp.zeros((LANES,), _f32)
                    hdh = h * dh
                    for dd in range(dh):
                        t = dd + iota
                        if dh & (dh - 1) == 0:
                            ddl = t & (dh - 1)
                        else:
                            ddl = jnp.where(t >= dh, t - dh, t)
                        cl = hdh + ddl
                        xlv = plsc.load_gather(xl_tile, [xl_base + cl])
                        xrv = plsc.load_gather(bufA, [rows, cl])
                        sv = xlv + xrv
                        sv = jnp.maximum(sv, sv * 0.2)
                        av = plsc.load_gather(att_v, [cl])
                        acc = acc + sv * av
                    a = jnp.exp(jnp.minimum(acc, 30.0))
                    a_buf[pl.ds(h * LANES, LANES)] = a
                    plsc.addupdate_scatter(denom_v, [dvb + h], a)
                    return 0
                lax.fori_loop(0, heads, hloop, 0)

                dvf = dv * d_feat

                def bloop(h, _):
                    av = a_buf[pl.ds(h * LANES, LANES)]
                    base = dvf + h * dh
                    for dd in range(dh):
                        t = dd + iota
                        if dh & (dh - 1) == 0:
                            ddl = t & (dh - 1)
                        else:
                            ddl = jnp.where(t >= dh, t - dh, t)
                        cl = h * dh + ddl
                        xv = plsc.load_gather(bufA, [rows, cl])
                        plsc.addupdate_scatter(out_un, [base + ddl], xv * av)
                    return 0
                lax.fori_loop(0, heads, bloop, 0)
                return 0
            lax.fori_loop(0, ngc, grp, 0)
            return 0
        lax.fori_loop(0, nch, chunk_body, 0)

        def norm(i, _):
            rb = i * d_feat
            for h in range(heads):
                dn = plsc.load_gather(
                    denom_v, [jnp.full((LANES,), i * heads + h, _i32)])
                rcp = 1.0 / (dn + 1e-16)
                for j in range(dh // LANES):
                    sl = pl.ds(rb + h * dh + j * LANES, LANES)
                    out_un[sl] = (out_un[sl] * rcp
                                  + b_v[pl.ds(h * dh + j * LANES, LANES)])
            return 0
        lax.fori_loop(0, NPR, norm, 0)

        pltpu.sync_copy(out_un.at[pl.ds(0, NPR * d_feat)],
                        out_hbm.at[pl.ds(lo * d_feat, NPR * d_feat)])
        return 0

    lax.fori_loop(0, 2, do_range, 0)


@functools.cache
def _gat_sc(heads, dh, d_store):
    d_feat = heads * dh
    return pl.kernel(
        functools.partial(_gat_body, heads, dh, d_store),
        out_type=jax.ShapeDtypeStruct((NPAD * d_feat,), _f32),
        mesh=_mesh(),
        compiler_params=_SC_PARAMS,
        scratch_types=[
            pltpu.VMEM((CAP,), _i32),
            pltpu.VMEM((CAP,), _i32),
            pltpu.VMEM(((NPR + 1) * d_feat,), _f32),
            pltpu.VMEM(((NPR + 1) * d_feat,), _f32),
            pltpu.VMEM((CHUNK, d_store), _f32),
            pltpu.VMEM((heads * LANES,), _f32),
            pltpu.VMEM((160 * heads,), _f32),
            pltpu.VMEM((d_feat,), _f32),
            pltpu.VMEM((d_feat,), _f32),
            pltpu.VMEM((128,), _i32),
            pltpu.SemaphoreType.DMA,
        ],
    )


def _mm2_body(elu, x_ref, wl_ref, wr_ref, ol_ref, or_ref):
    xv = x_ref[...]
    if elu:
        xv = jnp.where(xv > 0, xv, jnp.exp(jnp.minimum(xv, 0.0)) - 1.0)
    ol_ref[...] = jnp.dot(xv, wl_ref[...], preferred_element_type=_f32)
    or_ref[...] = jnp.dot(xv, wr_ref[...], preferred_element_type=_f32)


def _matmul2(x, wl, wr, elu, block_rows=1000):
    n, k = x.shape
    m = wl.shape[1]
    grid = (n // block_rows,)
    return pl.pallas_call(
        functools.partial(_mm2_body, elu),
        grid=grid,
        in_specs=[
            pl.BlockSpec((block_rows, k), lambda i: (i, 0)),
            pl.BlockSpec((k, m), lambda i: (0, 0)),
            pl.BlockSpec((k, m), lambda i: (0, 0)),
        ],
        out_specs=[
            pl.BlockSpec((block_rows, m), lambda i: (i, 0)),
            pl.BlockSpec((block_rows, m), lambda i: (i, 0)),
        ],
        out_shape=[
            jax.ShapeDtypeStruct((n, m), _f32),
            jax.ShapeDtypeStruct((n, m), _f32),
        ],
    )(x, wl, wr)


def _pool_body(h_ref, o_ref):
    pooled = jnp.sum(h_ref[...], axis=0, keepdims=True) / N
    colid = lax.broadcasted_iota(_i32, (1, 48), 1)
    mask = colid < NCLS
    pm = jnp.where(mask, pooled, -1e30)
    mx = jnp.max(pm, axis=1, keepdims=True)
    ex = jnp.where(mask, jnp.exp(pm - mx), 0.0)
    lse = jnp.log(jnp.sum(ex, axis=1, keepdims=True)) + mx
    o_ref[...] = jnp.broadcast_to(pooled - lse, (8, 48))


def _pool(h3):
    return pl.pallas_call(
        _pool_body,
        out_shape=jax.ShapeDtypeStruct((8, 48), _f32),
    )(h3)


def _pad_flat(xl, d_feat):
    return jnp.pad(xl.reshape(-1), (0, (NPAD + 1 - N) * d_feat))


def kernel(x, edge_index, Wl1, Wr1, att1, b1, Wl2, Wr2, att2, b2,
           Wl3, Wr3, att3, b3):
    src = edge_index[0]
    dst = edge_index[1]
    srcl, dstl, cnt = _compact(src, dst)

    xl1, xr1 = _matmul2(x, Wl1, Wr1, elu=False)
    h1 = _gat_sc(HEADS, HID, 256)(_pad_flat(xl1, 256), xr1, srcl, dstl, cnt,
                                  att1.reshape(-1), b1)
    h1 = h1.reshape(NPAD, HEADS * HID)[:N]

    xl2, xr2 = _matmul2(h1, Wl2, Wr2, elu=True)
    h2 = _gat_sc(HEADS, HID, 256)(_pad_flat(xl2, 256), xr2, srcl, dstl, cnt,
                                  att2.reshape(-1), b2)
    h2 = h2.reshape(NPAD, HEADS * HID)[:N]

    wl3 = jnp.pad(Wl3, ((0, 0), (0, 48 - NCLS)))
    wr3 = jnp.pad(Wr3, ((0, 0), (0, 128 - NCLS)))
    att3p = jnp.pad(att3, ((0, 0), (0, 48 - NCLS))).reshape(-1)
    b3p = jnp.pad(b3, (0, 48 - NCLS))
    xl3, xr3 = _matmul2(h2, jnp.pad(wl3, ((0, 0), (0, 80))), wr3, elu=True)
    h3 = _gat_sc(1, 48, 128)(_pad_flat(xl3[:, :48], 48), xr3, srcl, dstl,
                             cnt, att3p, b3p)
    h3 = h3.reshape(NPAD, 48)

    z = _pool(h3)
    return z[0:1, :NCLS]
